# baseline (speedup 1.0000x reference)
import functools

import jax
import jax.numpy as jnp
from jax.experimental import pallas as pl
from jax.experimental.pallas import tpu as pltpu
from jax.experimental.pallas import tpu_sc as plsc

N_CORES = 2
N_SUBCORES = 16
NW = N_CORES * N_SUBCORES
BATCH = 16384
D_MODEL = 128
CHUNK = 128
B_PER_W = BATCH // NW
N_CHUNKS = B_PER_W // CHUNK


def _make_kernel():
  mesh = plsc.VectorSubcoreMesh(
      core_axis_name="c", subcore_axis_name="s",
      num_cores=N_CORES, num_subcores=N_SUBCORES)
  out_type = (jax.ShapeDtypeStruct((BATCH, D_MODEL), jnp.float32),) * 3
  scratch = (
      [pltpu.VMEM((N_CHUNKS, CHUNK), jnp.int32)] * 3
      + [pltpu.VMEM((2 * CHUNK, D_MODEL), jnp.float32)] * 3
      + [pltpu.SemaphoreType.DMA] * 12
      + [pltpu.VMEM_SHARED((1000, D_MODEL), jnp.float32)]
  )

  @functools.partial(
      pl.kernel, out_type=out_type, mesh=mesh, scratch_types=scratch)
  def trans_e_gather(qe_h, qr_h, oe_h, ent_h, rel_h,
                     out_qe, out_qr, out_oe, *scr):
    idx_q, idx_r, idx_o = scr[0:3]
    pbufs = scr[3:6]
    gsems = scr[6:12]
    wsems = scr[12:15]
    isems = scr[15:18]
    rel_sh = scr[18]

    wid = jax.lax.axis_index("s") * N_CORES + jax.lax.axis_index("c")
    idx_base = wid * N_CHUNKS
    row_base = wid * B_PER_W

    icopies = [
        pltpu.async_copy(src.at[pl.ds(idx_base, N_CHUNKS)], dst, sem)
        for src, dst, sem in ((qe_h, idx_q, isems[0]),
                              (oe_h, idx_o, isems[1]),
                              (qr_h, idx_r, isems[2]))
    ]

    streams = ((idx_q, ent_h, out_qe), (idx_o, ent_h, out_oe),
               (idx_r, rel_sh, out_qr))

    def gather(k, p, half):
      idx_ref, tab, _ = streams[k]
      c = 2 * p + half
      return pltpu.async_copy(tab.at[idx_ref.at[c]],
                              pbufs[k].at[pl.ds(half * CHUNK, CHUNK)],
                              gsems[2 * k + half])

    def write(k, p):
      out = streams[k][2]
      return pltpu.async_copy(
          pbufs[k], out.at[pl.ds(row_base + 2 * p * CHUNK, 2 * CHUNK)],
          wsems[k])

    icopies[0].wait()
    icopies[1].wait()
    g = [gather(0, 0, 0), gather(0, 0, 1), gather(1, 0, 0), gather(1, 0, 1)]
    icopies[2].wait()

    @pl.when(jax.lax.axis_index("s") == 0)
    def _():
      pltpu.sync_copy(rel_h, rel_sh)
    plsc.subcore_barrier()
    g += [gather(2, 0, 0), gather(2, 0, 1)]

    w = {}
    for k in (0, 1, 2):
      g[2 * k].wait()
      g[2 * k + 1].wait()
      w[k] = write(k, 0)
    g2 = {}
    for k in (0, 1, 2):
      w[k].wait()
      g2[2 * k] = gather(k, 1, 0)
      g2[2 * k + 1] = gather(k, 1, 1)
    w2 = {}
    for k in (0, 1, 2):
      g2[2 * k].wait()
      g2[2 * k + 1].wait()
      w2[k] = write(k, 1)
    for k in (0, 1, 2):
      w2[k].wait()

  return trans_e_gather


_KERNEL = _make_kernel()


def kernel(query_entities, query_relations, obj_entities, ent_table, rel_table):
  qe = query_entities.reshape(NW * N_CHUNKS, CHUNK)
  qr = query_relations.reshape(NW * N_CHUNKS, CHUNK)
  oe = obj_entities.reshape(NW * N_CHUNKS, CHUNK)
  return _KERNEL(qe, qr, oe, ent_table, rel_table)


# speedup vs baseline: 1.0425x; 1.0425x over previous
import functools

import jax
import jax.numpy as jnp
from jax.experimental import pallas as pl
from jax.experimental.pallas import tpu as pltpu
from jax.experimental.pallas import tpu_sc as plsc

N_CORES = 2
N_SUBCORES = 16
NW = N_CORES * N_SUBCORES
BATCH = 16384
D_MODEL = 128
CHUNK = 128
B_PER_W = BATCH // NW
N_CHUNKS = B_PER_W // CHUNK
NBUF = 7


def _make_kernel():
  mesh = plsc.VectorSubcoreMesh(
      core_axis_name="c", subcore_axis_name="s",
      num_cores=N_CORES, num_subcores=N_SUBCORES)
  out_type = (jax.ShapeDtypeStruct((BATCH, D_MODEL), jnp.float32),) * 3
  scratch = (
      [pltpu.VMEM((N_CHUNKS, CHUNK), jnp.int32)] * 3
      + [pltpu.VMEM((CHUNK, D_MODEL), jnp.float32)] * NBUF
      + [pltpu.SemaphoreType.DMA] * (2 * NBUF + 3)
      + [pltpu.VMEM_SHARED((1000, D_MODEL), jnp.float32)]
  )

  @functools.partial(
      pl.kernel, out_type=out_type, mesh=mesh, scratch_types=scratch)
  def trans_e_gather(qe_h, qr_h, oe_h, ent_h, rel_h,
                     out_qe, out_qr, out_oe, *scr):
    idx_q, idx_r, idx_o = scr[0:3]
    bufs = scr[3:3 + NBUF]
    gsems = scr[3 + NBUF:3 + 2 * NBUF]
    osems = scr[3 + 2 * NBUF:3 + 3 * NBUF]
    isems = scr[3 + 3 * NBUF:3 + 3 * NBUF + 3]
    rel_sh = scr[3 + 3 * NBUF + 3]

    wid = jax.lax.axis_index("s") * N_CORES + jax.lax.axis_index("c")
    idx_base = wid * N_CHUNKS
    row_base = wid * B_PER_W

    icopies = [
        pltpu.async_copy(src.at[pl.ds(idx_base, N_CHUNKS)], dst, sem)
        for src, dst, sem in ((qe_h, idx_q, isems[0]),
                              (qr_h, idx_r, isems[1]),
                              (oe_h, idx_o, isems[2]))
    ]

    def task(idx_ref, c, tab, out):
      return (idx_ref.at[c], tab, out.at[pl.ds(row_base + c * CHUNK, CHUNK)])

    tasks = [
        task(idx_q, 0, ent_h, out_qe), task(idx_o, 0, ent_h, out_oe),
        task(idx_q, 1, ent_h, out_qe), task(idx_o, 1, ent_h, out_oe),
        task(idx_r, 0, rel_sh, out_qr), task(idx_q, 2, ent_h, out_qe),
        task(idx_r, 1, rel_sh, out_qr), task(idx_o, 2, ent_h, out_oe),
        task(idx_r, 2, rel_sh, out_qr), task(idx_q, 3, ent_h, out_qe),
        task(idx_r, 3, rel_sh, out_qr), task(idx_o, 3, ent_h, out_oe),
    ]
    nt = len(tasks)

    def start_gather(t):
      idx_s, tab, _ = tasks[t]
      return pltpu.async_copy(tab.at[idx_s], bufs[t % NBUF], gsems[t % NBUF])

    g = {}
    o = {}
    for ic in icopies:
      ic.wait()
    for t in range(4):
      g[t] = start_gather(t)
    @pl.when(jax.lax.axis_index("s") == 0)
    def _():
      pltpu.sync_copy(rel_h, rel_sh)
    plsc.subcore_barrier()
    for t in range(nt):
      g[t].wait()
      o[t] = pltpu.async_copy(bufs[t % NBUF], tasks[t][2], osems[t % NBUF])
      if t + 4 < nt:
        if t >= 3:
          o[t - 3].wait()
        g[t + 4] = start_gather(t + 4)
    for t in range(nt - 7, nt):
      o[t].wait()

  return trans_e_gather


_KERNEL = _make_kernel()


def kernel(query_entities, query_relations, obj_entities, ent_table, rel_table):
  qe = query_entities.reshape(NW * N_CHUNKS, CHUNK)
  qr = query_relations.reshape(NW * N_CHUNKS, CHUNK)
  oe = obj_entities.reshape(NW * N_CHUNKS, CHUNK)
  return _KERNEL(qe, qr, oe, ent_table, rel_table)
